# baseline (speedup 1.0000x reference)
import functools

import jax
import jax.numpy as jnp
from jax import lax
from jax.experimental import pallas as pl
from jax.experimental.pallas import tpu as pltpu
from jax.experimental.pallas import tpu_sc as plsc

N = 10000
E = 320000
B = 128
NTILES = 16
N_PAD = N + 8
STRIPE = 624
TAIL0 = NTILES * STRIPE
TAILN = N_PAD - TAIL0
RAW = 125
CHUNK = 128
SLAB = 20
RBLK = 1000
NBLK = N // RBLK
W = 128
DW = 16


def _mesh():
    return plsc.VectorSubcoreMesh(core_axis_name="c", subcore_axis_name="s")


def _pad5(idx, fill):
    a = idx.reshape(2, NTILES, -1, RAW)
    a = jnp.pad(a, ((0, 0), (0, 0), (0, 0), (0, CHUNK - RAW)),
                constant_values=fill)
    return a


def _stripe_zero(zeros_h, sp, s):
    r0 = s * STRIPE
    pltpu.sync_copy(zeros_h.at[pl.ds(r0, STRIPE)], sp.at[pl.ds(r0, STRIPE)])

    @pl.when(s == NTILES - 1)
    def _():
        pltpu.sync_copy(zeros_h.at[pl.ds(TAIL0, TAILN)],
                        sp.at[pl.ds(TAIL0, TAILN)])


def _stripe_drain(sp, out, c, s):
    r0 = s * STRIPE
    pltpu.sync_copy(sp.at[pl.ds(r0, STRIPE)], out.at[c, pl.ds(r0, STRIPE)])

    @pl.when(s == NTILES - 1)
    def _():
        pltpu.sync_copy(sp.at[pl.ds(TAIL0, TAILN)],
                        out.at[c, pl.ds(TAIL0, TAILN)])


def _make_deg_kernel():
    n_chunks = E // 2 // NTILES // RAW

    @functools.partial(
        pl.kernel,
        mesh=_mesh(),
        out_type=jax.ShapeDtypeStruct((2, N_PAD, DW), jnp.float32),
        scratch_types=[
            pltpu.VMEM((CHUNK,), jnp.int32),
            pltpu.VMEM((CHUNK, DW), jnp.float32),
            pltpu.VMEM_SHARED((N_PAD, DW), jnp.float32),
        ],
    )
    def deg_kernel(dst4, ones_h, zeros_h, out, idx1, ones_v, deg_sp):
        c = lax.axis_index("c")
        s = lax.axis_index("s")
        _stripe_zero(zeros_h, deg_sp, s)
        pltpu.sync_copy(ones_h, ones_v)
        plsc.subcore_barrier()

        def chunk(k, carry):
            pltpu.sync_copy(dst4.at[c, s, k], idx1)
            pltpu.sync_copy(ones_v, deg_sp.at[idx1], add=True)
            return carry

        lax.fori_loop(0, n_chunks, chunk, 0)
        plsc.subcore_barrier()
        _stripe_drain(deg_sp, out, c, s)

    return deg_kernel


def _make_lap_kernel(n_chunks):
    @functools.partial(
        pl.kernel,
        mesh=_mesh(),
        out_type=jax.ShapeDtypeStruct((2, N_PAD, W), jnp.float32),
        scratch_types=[
            pltpu.VMEM((CHUNK,), jnp.int32),
            pltpu.VMEM((CHUNK,), jnp.int32),
            pltpu.VMEM((CHUNK, W), jnp.float32),
            pltpu.VMEM_SHARED((N_PAD, W), jnp.float32),
        ],
    )
    def lap_kernel(src4, dst4, xs2, zeros_h, out, sidx, didx, rows_v, agg_sp):
        c = lax.axis_index("c")
        s = lax.axis_index("s")
        _stripe_zero(zeros_h, agg_sp, s)
        plsc.subcore_barrier()

        def chunk(k, carry):
            pltpu.sync_copy(src4.at[c, s, k], sidx)
            pltpu.sync_copy(dst4.at[c, s, k], didx)
            pltpu.sync_copy(xs2.at[sidx], rows_v)
            pltpu.sync_copy(rows_v, agg_sp.at[didx], add=True)
            return carry

        lax.fori_loop(0, n_chunks, chunk, 0)
        plsc.subcore_barrier()
        _stripe_drain(agg_sp, out, c, s)

    return lap_kernel


def _prep_body(sig_ref, deg_ref, emb_ref, h0_ref, xs_ref, dinv_ref):
    sig = sig_ref[0, 0, :]
    oh = (sig[:, None] == lax.broadcasted_iota(jnp.int32, (RBLK, 32), 1))
    h0 = jnp.dot(oh.astype(jnp.float32), emb_ref[...],
                 preferred_element_type=jnp.float32)
    deg = deg_ref[0][:, :8] + deg_ref[1][:, :8]
    dinv8 = lax.rsqrt(jnp.maximum(deg, 1.0))
    h0_ref[...] = h0
    dinv_ref[...] = dinv8
    xs_ref[...] = h0 * dinv8[:, 0:1]


def _prep_tc(signal3, deg8, emb_pad):
    f = 128
    return pl.pallas_call(
        _prep_body,
        grid=(NBLK,),
        in_specs=[
            pl.BlockSpec((1, 1, RBLK), lambda i: (i, 0, 0)),
            pl.BlockSpec((2, RBLK, DW), lambda i: (0, i, 0)),
            pl.BlockSpec((32, f), lambda i: (0, 0)),
        ],
        out_specs=[
            pl.BlockSpec((RBLK, f), lambda i: (i, 0)),
            pl.BlockSpec((RBLK, f), lambda i: (i, 0)),
            pl.BlockSpec((RBLK, 8), lambda i: (i, 0)),
        ],
        out_shape=[
            jax.ShapeDtypeStruct((N, f), jnp.float32),
            jax.ShapeDtypeStruct((N, f), jnp.float32),
            jax.ShapeDtypeStruct((N, 8), jnp.float32),
        ],
    )(signal3, deg8, emb_pad)


def _combine(agg_ref, feat_split):
    if feat_split:
        return jnp.concatenate([agg_ref[0], agg_ref[1]], axis=-1)
    return agg_ref[0] + agg_ref[1]


def _split_store(xs_ref, xs, feat_split):
    if feat_split:
        h = xs.shape[1] // 2
        xs_ref[0] = xs[:, :h]
        xs_ref[1] = xs[:, h:]
    else:
        xs_ref[...] = xs


def _lap_a_body(h_ref, agg_ref, dinv_ref, lam_ref, t1_ref, xs_ref, *, fs_in):
    h = h_ref[...]
    agg = _combine(agg_ref, fs_in)
    dv = dinv_ref[:, 0:1]
    rn = 2.0 / lam_ref[0, 0]
    t1 = rn * (h - agg * dv) - h
    t1_ref[...] = t1
    _split_store(xs_ref, t1 * dv, fs_in)


def _lap_a_tc(h, agg, dinv8, lam):
    f = h.shape[1]
    fs = f == 256
    xs_spec = (pl.BlockSpec((2, RBLK, W), lambda i: (0, i, 0)) if fs
               else pl.BlockSpec((RBLK, f), lambda i: (i, 0)))
    xs_shape = (jax.ShapeDtypeStruct((2, N, W), jnp.float32) if fs
                else jax.ShapeDtypeStruct((N, f), jnp.float32))
    return pl.pallas_call(
        functools.partial(_lap_a_body, fs_in=fs),
        grid=(NBLK,),
        in_specs=[
            pl.BlockSpec((RBLK, f), lambda i: (i, 0)),
            pl.BlockSpec((2, RBLK, W), lambda i: (0, i, 0)),
            pl.BlockSpec((RBLK, 8), lambda i: (i, 0)),
            pl.BlockSpec((1, 1), lambda i: (0, 0)),
        ],
        out_specs=[pl.BlockSpec((RBLK, f), lambda i: (i, 0)), xs_spec],
        out_shape=[jax.ShapeDtypeStruct((N, f), jnp.float32), xs_shape],
    )(h, agg, dinv8, lam)


def _lap_b_body(h_ref, t1_ref, agg_ref, dinv_ref, lam_ref, w0_ref, w1_ref,
                w2_ref, b_ref, o_ref, xs_ref, *, fs_in, fs_out):
    h = h_ref[...]
    t1 = t1_ref[...]
    agg = _combine(agg_ref, fs_in)
    dv = dinv_ref[:, 0:1]
    rn = 2.0 / lam_ref[0, 0]
    t2 = 2.0 * (rn * (t1 - agg * dv) - t1) - h
    out = (jnp.dot(h, w0_ref[...], preferred_element_type=jnp.float32)
           + jnp.dot(t1, w1_ref[...], preferred_element_type=jnp.float32)
           + jnp.dot(t2, w2_ref[...], preferred_element_type=jnp.float32)
           + b_ref[...])
    o_ref[...] = out
    if xs_ref is not None:
        _split_store(xs_ref, out * dv, fs_out)


def _lap_b_tc(h, t1, agg, dinv8, lam, w0, w1, w2, b, with_xs):
    f = h.shape[1]
    fo = w0.shape[1]
    fs_in = f == 256
    fs_out = fo == 256
    out_specs = [pl.BlockSpec((RBLK, fo), lambda i: (i, 0))]
    out_shape = [jax.ShapeDtypeStruct((N, fo), jnp.float32)]
    if with_xs:
        if fs_out:
            out_specs.append(pl.BlockSpec((2, RBLK, W), lambda i: (0, i, 0)))
            out_shape.append(jax.ShapeDtypeStruct((2, N, W), jnp.float32))
        else:
            out_specs.append(pl.BlockSpec((RBLK, fo), lambda i: (i, 0)))
            out_shape.append(jax.ShapeDtypeStruct((N, fo), jnp.float32))

    if with_xs:
        body = functools.partial(_lap_b_body, fs_in=fs_in, fs_out=fs_out)
    else:
        def body(*refs):
            _lap_b_body(*refs, None, fs_in=fs_in, fs_out=fs_out)

    res = pl.pallas_call(
        body,
        grid=(NBLK,),
        in_specs=[
            pl.BlockSpec((RBLK, f), lambda i: (i, 0)),
            pl.BlockSpec((RBLK, f), lambda i: (i, 0)),
            pl.BlockSpec((2, RBLK, W), lambda i: (0, i, 0)),
            pl.BlockSpec((RBLK, 8), lambda i: (i, 0)),
            pl.BlockSpec((1, 1), lambda i: (0, 0)),
            pl.BlockSpec((f, fo), lambda i: (0, 0)),
            pl.BlockSpec((f, fo), lambda i: (0, 0)),
            pl.BlockSpec((f, fo), lambda i: (0, 0)),
            pl.BlockSpec((1, fo), lambda i: (0, 0)),
        ],
        out_specs=out_specs,
        out_shape=out_shape,
    )(h, t1, agg, dinv8, lam, w0, w1, w2, b)
    if with_xs:
        return res[0], res[1]
    return res[0], None


def _readout_body(gid_ref, h_ref, w1_ref, b1_ref, w2_ref, b2_ref, o_ref, acc):
    i = pl.program_id(0)

    @pl.when(i == 0)
    def _():
        acc[...] = jnp.zeros_like(acc)

    g = gid_ref[0, 0, :]
    oh = (g[:, None] == lax.broadcasted_iota(jnp.int32, (RBLK, B), 1))
    acc[...] += lax.dot_general(oh.astype(jnp.float32), h_ref[...],
                                (((0,), (0,)), ((), ())),
                                preferred_element_type=jnp.float32)

    @pl.when(i == NBLK - 1)
    def _():
        z = jnp.maximum(jnp.dot(acc[...], w1_ref[...],
                                preferred_element_type=jnp.float32)
                        + b1_ref[...], 0.0)
        o_ref[...] = jnp.dot(z, w2_ref[...],
                             preferred_element_type=jnp.float32) + b2_ref[...]


def _readout_tc(gid3, h, w1, b1, w2, b2):
    f = h.shape[1]
    ncls = w2.shape[1]
    return pl.pallas_call(
        _readout_body,
        grid=(NBLK,),
        in_specs=[
            pl.BlockSpec((1, 1, RBLK), lambda i: (i, 0, 0)),
            pl.BlockSpec((RBLK, f), lambda i: (i, 0)),
            pl.BlockSpec((f, f), lambda i: (0, 0)),
            pl.BlockSpec((1, f), lambda i: (0, 0)),
            pl.BlockSpec((f, ncls), lambda i: (0, 0)),
            pl.BlockSpec((1, ncls), lambda i: (0, 0)),
        ],
        out_specs=pl.BlockSpec((B, ncls), lambda i: (0, 0)),
        out_shape=jax.ShapeDtypeStruct((B, ncls), jnp.float32),
        scratch_shapes=[pltpu.VMEM((B, f), jnp.float32)],
    )(gid3, h, w1, b1, w2, b2)


def _cheb_jax(h, src, dst, d_invsqrt, re_norm, Wk, b, lap_sc=None):
    n = h.shape[0]

    def lap_agg(x):
        y = (x * d_invsqrt)[src]
        return jax.ops.segment_sum(y, dst, num_segments=n)

    def lap_apply(x, agg):
        return re_norm * (x - agg * d_invsqrt) - x

    out = h @ Wk[0]
    t1 = lap_apply(h, lap_agg(h) if lap_sc is None else lap_sc(h * d_invsqrt))
    out = out + t1 @ Wk[1]
    t2 = 2.0 * lap_apply(
        t1, lap_agg(t1) if lap_sc is None else lap_sc(t1 * d_invsqrt)) - h
    out = out + t2 @ Wk[2]
    return out + b


def _kernel_debug(signal, edge_index, node_graph_id, lambda_max, emb, W1, b1, W2, b2,
           W3, b3, mlp_w1, mlp_b1, mlp_w2, mlp_b2,
           use_sc_deg=False, use_sc_lap1=False):
    src = edge_index[0].astype(jnp.int32)
    dst = edge_index[1].astype(jnp.int32)
    dst5_e = _pad5(dst, N)
    src5_e = _pad5(src, 0)
    ones8 = jnp.ones((CHUNK, DW), jnp.float32)
    zeros8 = jnp.zeros((N_PAD, DW), jnp.float32)
    zeros = jnp.zeros((N_PAD, W), jnp.float32)

    if use_sc_deg:
        deg_kernel = _make_deg_kernel()
        deg8 = deg_kernel(dst5_e, ones8, zeros8)
        deg = deg8[0, :N, 0] + deg8[1, :N, 0]
    else:
        deg = jax.ops.segment_sum(jnp.ones((E,), jnp.float32), dst,
                                  num_segments=N)

    lap_sc_128 = None
    if use_sc_lap1:
        lap_e = _make_lap_kernel(E // 2 // NTILES // RAW)

        def lap_sc_128(xs):
            p = lap_e(src5_e, dst5_e, xs, zeros)
            return p[0, :N] + p[1, :N]

    d_invsqrt = jnp.power(jnp.clip(deg, 1.0, None), -0.5)[:, None]
    re_norm = 2.0 / lambda_max[0]
    h = jnp.take(emb, signal, axis=0)
    h = _cheb_jax(h, src, dst, d_invsqrt, re_norm, W1, b1, lap_sc=lap_sc_128)
    h = _cheb_jax(h, src, dst, d_invsqrt, re_norm, W2, b2)
    h = _cheb_jax(h, src, dst, d_invsqrt, re_norm, W3, b3)
    hg = jax.ops.segment_sum(h, node_graph_id, num_segments=B)
    return jax.nn.relu(hg @ mlp_w1 + mlp_b1) @ mlp_w2 + mlp_b2


def _make_zero_drain_kernel():
    @functools.partial(
        pl.kernel,
        mesh=_mesh(),
        out_type=jax.ShapeDtypeStruct((2, N_PAD, DW), jnp.float32),
        scratch_types=[pltpu.VMEM_SHARED((N_PAD, DW), jnp.float32)],
    )
    def k(zeros_h, out, sp):
        c = lax.axis_index("c")
        s = lax.axis_index("s")
        _stripe_zero(zeros_h, sp, s)
        plsc.subcore_barrier()
        _stripe_drain(sp, out, c, s)

    return k


def _make_idx_rt_kernel(n_chunks):
    @functools.partial(
        pl.kernel,
        mesh=_mesh(),
        out_type=jax.ShapeDtypeStruct((2, NTILES, n_chunks, CHUNK),
                                      jnp.int32),
        scratch_types=[pltpu.VMEM((CHUNK,), jnp.int32)],
    )
    def k(dst4, out, v):
        c = lax.axis_index("c")
        s = lax.axis_index("s")

        def chunk(kk, carry):
            pltpu.sync_copy(dst4.at[c, s, kk], v)
            pltpu.sync_copy(v, out.at[c, s, kk])
            return carry

        lax.fori_loop(0, n_chunks, chunk, 0)

    return k


def _kernel_diag(signal, edge_index, node_graph_id, lambda_max, emb, W1, b1,
                 W2, b2, W3, b3, mlp_w1, mlp_b1, mlp_w2, mlp_b2):
    dst = edge_index[1].astype(jnp.int32)
    dst5_e = _pad5(dst, N)
    ones8 = jnp.ones((CHUNK, DW), jnp.float32)
    zeros8 = jnp.zeros((N_PAD, DW), jnp.float32)

    @functools.partial(
        pl.kernel,
        mesh=_mesh(),
        out_type=jax.ShapeDtypeStruct((2, N_PAD, DW), jnp.float32),
        scratch_types=[
            pltpu.VMEM((CHUNK,), jnp.int32),
            pltpu.VMEM((CHUNK, DW), jnp.float32),
            pltpu.VMEM((STRIPE, DW), jnp.float32),
            pltpu.VMEM_SHARED((N_PAD, DW), jnp.float32),
        ],
    )
    def probe(idx_h, vals_h, zeros_h, out, idx1, vals_v, dummy_v, sp):
        c = lax.axis_index("c")
        s = lax.axis_index("s")
        _stripe_zero(zeros_h, sp, s)
        plsc.subcore_barrier()

        @pl.when((c == 0) & (s == 0))
        def _():
            pltpu.sync_copy(idx_h, idx1)
            pltpu.sync_copy(vals_h, vals_v)
            pltpu.sync_copy(vals_v, sp.at[idx1], add=True)

        def delay(j, carry):
            pltpu.sync_copy(zeros_h.at[pl.ds(8000, STRIPE)], dummy_v)
            return carry

        lax.fori_loop(0, 100, delay, 0)
        plsc.subcore_barrier()
        _stripe_drain(sp, out, c, s)

    idxp = (jnp.arange(CHUNK, dtype=jnp.int32) * 5 + 3)
    vals = jnp.broadcast_to(
        (jnp.arange(CHUNK, dtype=jnp.float32) + 1.0)[:, None],
        (CHUNK, DW)).astype(jnp.float32)
    po = probe(idxp, vals, zeros8)
    out0 = po[0, :N, 0]
    rows = jnp.arange(N, dtype=jnp.float32)
    r1 = jnp.sum(rows * (jnp.abs(out0 - 1.0) < 0.5))
    cnt = jnp.sum((out0 != 0.0).astype(jnp.float32))
    diag = jnp.clip(cnt, 0.0, 999.0) + 1000.0 * jnp.clip(r1, 0.0, 9999.0)
    return jnp.full((B, 1), diag, jnp.float32)


def _prep2_body(sig_ref, deg_ref, emb_ref, h0_ref, xs_ref, dinv_ref):
    sig = sig_ref[0, 0, :]
    oh = (sig[:, None] == lax.broadcasted_iota(jnp.int32, (RBLK, 32), 1))
    h0 = jnp.dot(oh.astype(jnp.float32), emb_ref[...],
                 preferred_element_type=jnp.float32)
    dinv8 = lax.rsqrt(jnp.maximum(deg_ref[...], 1.0))
    h0_ref[...] = h0
    dinv_ref[...] = dinv8
    xs_ref[...] = h0 * dinv8[:, 0:1]


def _prep2_tc(signal3, deg8, emb_pad):
    f = 128
    return pl.pallas_call(
        _prep2_body,
        grid=(NBLK,),
        in_specs=[
            pl.BlockSpec((1, 1, RBLK), lambda i: (i, 0, 0)),
            pl.BlockSpec((RBLK, 8), lambda i: (i, 0)),
            pl.BlockSpec((32, f), lambda i: (0, 0)),
        ],
        out_specs=[
            pl.BlockSpec((RBLK, f), lambda i: (i, 0)),
            pl.BlockSpec((RBLK, f), lambda i: (i, 0)),
            pl.BlockSpec((RBLK, 8), lambda i: (i, 0)),
        ],
        out_shape=[
            jax.ShapeDtypeStruct((N, f), jnp.float32),
            jax.ShapeDtypeStruct((N, f), jnp.float32),
            jax.ShapeDtypeStruct((N, 8), jnp.float32),
        ],
    )(signal3, deg8, emb_pad)


def _lap_a2_body(h_ref, agg_ref, dinv_ref, lam_ref, t1_ref, xs_ref):
    h = h_ref[...]
    dv = dinv_ref[:, 0:1]
    rn = 2.0 / lam_ref[0, 0]
    t1 = rn * (h - agg_ref[...] * dv) - h
    t1_ref[...] = t1
    xs_ref[...] = t1 * dv


def _lap_a2_tc(h, agg, dinv8, lam):
    f = h.shape[1]
    return pl.pallas_call(
        _lap_a2_body,
        grid=(NBLK,),
        in_specs=[
            pl.BlockSpec((RBLK, f), lambda i: (i, 0)),
            pl.BlockSpec((RBLK, f), lambda i: (i, 0)),
            pl.BlockSpec((RBLK, 8), lambda i: (i, 0)),
            pl.BlockSpec((1, 1), lambda i: (0, 0)),
        ],
        out_specs=[pl.BlockSpec((RBLK, f), lambda i: (i, 0)),
                   pl.BlockSpec((RBLK, f), lambda i: (i, 0))],
        out_shape=[jax.ShapeDtypeStruct((N, f), jnp.float32),
                   jax.ShapeDtypeStruct((N, f), jnp.float32)],
    )(h, agg, dinv8, lam)


def _lap_b2_body(h_ref, t1_ref, agg_ref, dinv_ref, lam_ref, w0_ref, w1_ref,
                 w2_ref, b_ref, o_ref, xs_ref):
    h = h_ref[...]
    t1 = t1_ref[...]
    dv = dinv_ref[:, 0:1]
    rn = 2.0 / lam_ref[0, 0]
    t2 = 2.0 * (rn * (t1 - agg_ref[...] * dv) - t1) - h
    out = (jnp.dot(h, w0_ref[...], preferred_element_type=jnp.float32)
           + jnp.dot(t1, w1_ref[...], preferred_element_type=jnp.float32)
           + jnp.dot(t2, w2_ref[...], preferred_element_type=jnp.float32)
           + b_ref[...])
    o_ref[...] = out
    if xs_ref is not None:
        xs_ref[...] = out * dv


def _lap_b2_tc(h, t1, agg, dinv8, lam, w0, w1, w2, b, with_xs):
    f = h.shape[1]
    fo = w0.shape[1]
    out_specs = [pl.BlockSpec((RBLK, fo), lambda i: (i, 0))]
    out_shape = [jax.ShapeDtypeStruct((N, fo), jnp.float32)]
    if with_xs:
        out_specs.append(pl.BlockSpec((RBLK, fo), lambda i: (i, 0)))
        out_shape.append(jax.ShapeDtypeStruct((N, fo), jnp.float32))
        body = _lap_b2_body
    else:
        def body(*refs):
            _lap_b2_body(*refs, None)

    res = pl.pallas_call(
        body,
        grid=(NBLK,),
        in_specs=[
            pl.BlockSpec((RBLK, f), lambda i: (i, 0)),
            pl.BlockSpec((RBLK, f), lambda i: (i, 0)),
            pl.BlockSpec((RBLK, f), lambda i: (i, 0)),
            pl.BlockSpec((RBLK, 8), lambda i: (i, 0)),
            pl.BlockSpec((1, 1), lambda i: (0, 0)),
            pl.BlockSpec((f, fo), lambda i: (0, 0)),
            pl.BlockSpec((f, fo), lambda i: (0, 0)),
            pl.BlockSpec((f, fo), lambda i: (0, 0)),
            pl.BlockSpec((1, fo), lambda i: (0, 0)),
        ],
        out_specs=out_specs,
        out_shape=out_shape,
    )(h, t1, agg, dinv8, lam, w0, w1, w2, b)
    if with_xs:
        return res[0], res[1]
    return res[0], None


def kernel(signal, edge_index, node_graph_id, lambda_max, emb, W1, b1, W2, b2,
           W3, b3, mlp_w1, mlp_b1, mlp_w2, mlp_b2):
    src = edge_index[0].astype(jnp.int32)
    dst = edge_index[1].astype(jnp.int32)

    deg = jax.ops.segment_sum(jnp.ones((E,), jnp.float32), dst,
                              num_segments=N)
    deg8 = jnp.broadcast_to(deg[:, None], (N, 8))

    def lap_agg(xs):
        return jax.ops.segment_sum(jnp.take(xs, src, axis=0), dst,
                                   num_segments=N)

    signal3 = signal.astype(jnp.int32).reshape(NBLK, 1, RBLK)
    gid3 = node_graph_id.astype(jnp.int32).reshape(NBLK, 1, RBLK)
    emb_pad = jnp.zeros((32, 128), jnp.float32).at[:28].set(emb)
    lam = lambda_max.reshape(1, 1)

    h, xs, dinv8 = _prep2_tc(signal3, deg8, emb_pad)

    for Wk, bk in ((W1, b1), (W2, b2), (W3, b3)):
        t1, xs1 = _lap_a2_tc(h, lap_agg(xs), dinv8, lam)
        last = Wk is W3
        h, xs = _lap_b2_tc(h, t1, lap_agg(xs1), dinv8, lam, Wk[0], Wk[1],
                           Wk[2], bk.reshape(1, -1), with_xs=not last)

    return _readout_tc(gid3, h, mlp_w1, mlp_b1.reshape(1, -1), mlp_w2,
                       mlp_b2.reshape(1, -1))


def _kernel_full(signal, edge_index, node_graph_id, lambda_max, emb, W1, b1, W2, b2,
           W3, b3, mlp_w1, mlp_b1, mlp_w2, mlp_b2):
    src = edge_index[0].astype(jnp.int32)
    dst = edge_index[1].astype(jnp.int32)

    dst4_e = _pad5(dst, N)
    src4_e = _pad5(src, 0)
    src4_f = _pad5(jnp.concatenate([src, src + N]), 0)
    dst4_f = _pad5(jnp.concatenate([dst, dst]), N)

    ones8 = jnp.ones((CHUNK, DW), jnp.float32)
    zeros8 = jnp.zeros((N_PAD, DW), jnp.float32)
    zeros = jnp.zeros((N_PAD, W), jnp.float32)

    deg_kernel = _make_deg_kernel()
    lap_e = _make_lap_kernel(E // 2 // NTILES // RAW)
    lap_f = _make_lap_kernel(E // NTILES // RAW)

    deg8 = deg_kernel(dst4_e, ones8, zeros8)

    signal3 = signal.astype(jnp.int32).reshape(NBLK, 1, RBLK)
    gid3 = node_graph_id.astype(jnp.int32).reshape(NBLK, 1, RBLK)
    emb_pad = jnp.zeros((32, 128), jnp.float32).at[:28].set(emb)
    lam = lambda_max.reshape(1, 1)

    h, xs, dinv8 = _prep_tc(signal3, deg8, emb_pad)

    for Wk, bk in ((W1, b1), (W2, b2), (W3, b3)):
        fs = h.shape[1] == 256
        lap = lap_f if fs else lap_e
        srcx, dstx = (src4_f, dst4_f) if fs else (src4_e, dst4_e)
        xs2 = xs.reshape(-1, W)
        agg1 = lap(srcx, dstx, xs2, zeros)
        t1, xs1 = _lap_a_tc(h, agg1, dinv8, lam)
        agg2 = lap(srcx, dstx, xs1.reshape(-1, W), zeros)
        last = Wk is W3
        h, xs = _lap_b_tc(h, t1, agg2, dinv8, lam, Wk[0], Wk[1], Wk[2],
                          bk.reshape(1, -1), with_xs=not last)

    return _readout_tc(gid3, h, mlp_w1, mlp_b1.reshape(1, -1), mlp_w2,
                       mlp_b2.reshape(1, -1))


# speedup vs baseline: 1.0000x; 1.0000x over previous
import jax
import jax.numpy as jnp
from jax import lax
from jax.experimental import pallas as pl
from jax.experimental.pallas import tpu as pltpu

N = 10000
E = 320000
B = 128
RBLK = 1000
NBLK = N // RBLK


def _prep_body(sig_ref, deg_ref, emb_ref, h0_ref, xs_ref, dinv_ref):
    sig = sig_ref[0, 0, :]
    oh = (sig[:, None] == lax.broadcasted_iota(jnp.int32, (RBLK, 32), 1))
    h0 = jnp.dot(oh.astype(jnp.float32), emb_ref[...],
                 preferred_element_type=jnp.float32)
    dinv8 = lax.rsqrt(jnp.maximum(deg_ref[...], 1.0))
    h0_ref[...] = h0
    dinv_ref[...] = dinv8
    xs_ref[...] = h0 * dinv8[:, 0:1]


def _prep_tc(signal3, deg8, emb_pad):
    f = 128
    return pl.pallas_call(
        _prep_body,
        grid=(NBLK,),
        in_specs=[
            pl.BlockSpec((1, 1, RBLK), lambda i: (i, 0, 0)),
            pl.BlockSpec((RBLK, 8), lambda i: (i, 0)),
            pl.BlockSpec((32, f), lambda i: (0, 0)),
        ],
        out_specs=[
            pl.BlockSpec((RBLK, f), lambda i: (i, 0)),
            pl.BlockSpec((RBLK, f), lambda i: (i, 0)),
            pl.BlockSpec((RBLK, 8), lambda i: (i, 0)),
        ],
        out_shape=[
            jax.ShapeDtypeStruct((N, f), jnp.float32),
            jax.ShapeDtypeStruct((N, f), jnp.float32),
            jax.ShapeDtypeStruct((N, 8), jnp.float32),
        ],
    )(signal3, deg8, emb_pad)


def _lap_a_body(h_ref, agg_ref, dinv_ref, lam_ref, t1_ref, xs_ref):
    h = h_ref[...]
    dv = dinv_ref[:, 0:1]
    rn = 2.0 / lam_ref[0, 0]
    t1 = rn * (h - agg_ref[...] * dv) - h
    t1_ref[...] = t1
    xs_ref[...] = t1 * dv


def _lap_a_tc(h, agg, dinv8, lam):
    f = h.shape[1]
    return pl.pallas_call(
        _lap_a_body,
        grid=(NBLK,),
        in_specs=[
            pl.BlockSpec((RBLK, f), lambda i: (i, 0)),
            pl.BlockSpec((RBLK, f), lambda i: (i, 0)),
            pl.BlockSpec((RBLK, 8), lambda i: (i, 0)),
            pl.BlockSpec((1, 1), lambda i: (0, 0)),
        ],
        out_specs=[pl.BlockSpec((RBLK, f), lambda i: (i, 0)),
                   pl.BlockSpec((RBLK, f), lambda i: (i, 0))],
        out_shape=[jax.ShapeDtypeStruct((N, f), jnp.float32),
                   jax.ShapeDtypeStruct((N, f), jnp.float32)],
    )(h, agg, dinv8, lam)


def _lap_b_body(h_ref, t1_ref, agg_ref, dinv_ref, lam_ref, w0_ref, w1_ref,
                w2_ref, b_ref, o_ref, xs_ref):
    h = h_ref[...]
    t1 = t1_ref[...]
    dv = dinv_ref[:, 0:1]
    rn = 2.0 / lam_ref[0, 0]
    t2 = 2.0 * (rn * (t1 - agg_ref[...] * dv) - t1) - h
    out = (jnp.dot(h, w0_ref[...], preferred_element_type=jnp.float32)
           + jnp.dot(t1, w1_ref[...], preferred_element_type=jnp.float32)
           + jnp.dot(t2, w2_ref[...], preferred_element_type=jnp.float32)
           + b_ref[...])
    o_ref[...] = out
    if xs_ref is not None:
        xs_ref[...] = out * dv


def _lap_b_tc(h, t1, agg, dinv8, lam, w0, w1, w2, b, with_xs):
    f = h.shape[1]
    fo = w0.shape[1]
    out_specs = [pl.BlockSpec((RBLK, fo), lambda i: (i, 0))]
    out_shape = [jax.ShapeDtypeStruct((N, fo), jnp.float32)]
    if with_xs:
        out_specs.append(pl.BlockSpec((RBLK, fo), lambda i: (i, 0)))
        out_shape.append(jax.ShapeDtypeStruct((N, fo), jnp.float32))
        body = _lap_b_body
    else:
        def body(*refs):
            _lap_b_body(*refs, None)

    res = pl.pallas_call(
        body,
        grid=(NBLK,),
        in_specs=[
            pl.BlockSpec((RBLK, f), lambda i: (i, 0)),
            pl.BlockSpec((RBLK, f), lambda i: (i, 0)),
            pl.BlockSpec((RBLK, f), lambda i: (i, 0)),
            pl.BlockSpec((RBLK, 8), lambda i: (i, 0)),
            pl.BlockSpec((1, 1), lambda i: (0, 0)),
            pl.BlockSpec((f, fo), lambda i: (0, 0)),
            pl.BlockSpec((f, fo), lambda i: (0, 0)),
            pl.BlockSpec((f, fo), lambda i: (0, 0)),
            pl.BlockSpec((1, fo), lambda i: (0, 0)),
        ],
        out_specs=out_specs,
        out_shape=out_shape,
    )(h, t1, agg, dinv8, lam, w0, w1, w2, b)
    if with_xs:
        return res[0], res[1]
    return res[0], None


def _readout_body(gid_ref, h_ref, w1_ref, b1_ref, w2_ref, b2_ref, o_ref, acc):
    i = pl.program_id(0)

    @pl.when(i == 0)
    def _():
        acc[...] = jnp.zeros_like(acc)

    g = gid_ref[0, 0, :]
    oh = (g[:, None] == lax.broadcasted_iota(jnp.int32, (RBLK, B), 1))
    acc[...] += lax.dot_general(oh.astype(jnp.float32), h_ref[...],
                                (((0,), (0,)), ((), ())),
                                preferred_element_type=jnp.float32)

    @pl.when(i == NBLK - 1)
    def _():
        z = jnp.maximum(jnp.dot(acc[...], w1_ref[...],
                                preferred_element_type=jnp.float32)
                        + b1_ref[...], 0.0)
        o_ref[...] = jnp.dot(z, w2_ref[...],
                             preferred_element_type=jnp.float32) + b2_ref[...]


def _readout_tc(gid3, h, w1, b1, w2, b2):
    f = h.shape[1]
    ncls = w2.shape[1]
    return pl.pallas_call(
        _readout_body,
        grid=(NBLK,),
        in_specs=[
            pl.BlockSpec((1, 1, RBLK), lambda i: (i, 0, 0)),
            pl.BlockSpec((RBLK, f), lambda i: (i, 0)),
            pl.BlockSpec((f, f), lambda i: (0, 0)),
            pl.BlockSpec((1, f), lambda i: (0, 0)),
            pl.BlockSpec((f, ncls), lambda i: (0, 0)),
            pl.BlockSpec((1, ncls), lambda i: (0, 0)),
        ],
        out_specs=pl.BlockSpec((B, ncls), lambda i: (0, 0)),
        out_shape=jax.ShapeDtypeStruct((B, ncls), jnp.float32),
        scratch_shapes=[pltpu.VMEM((B, f), jnp.float32)],
    )(gid3, h, w1, b1, w2, b2)


def kernel(signal, edge_index, node_graph_id, lambda_max, emb, W1, b1, W2, b2,
           W3, b3, mlp_w1, mlp_b1, mlp_w2, mlp_b2):
    src = edge_index[0].astype(jnp.int32)
    dst = edge_index[1].astype(jnp.int32)

    deg = jax.ops.segment_sum(jnp.ones((E,), jnp.float32), dst,
                              num_segments=N)
    deg8 = jnp.broadcast_to(deg[:, None], (N, 8))

    def lap_agg(xs):
        return jax.ops.segment_sum(jnp.take(xs, src, axis=0), dst,
                                   num_segments=N)

    signal3 = signal.astype(jnp.int32).reshape(NBLK, 1, RBLK)
    gid3 = node_graph_id.astype(jnp.int32).reshape(NBLK, 1, RBLK)
    emb_pad = jnp.zeros((32, 128), jnp.float32).at[:28].set(emb)
    lam = lambda_max.reshape(1, 1)

    h, xs, dinv8 = _prep_tc(signal3, deg8, emb_pad)

    for Wk, bk in ((W1, b1), (W2, b2), (W3, b3)):
        t1, xs1 = _lap_a_tc(h, lap_agg(xs), dinv8, lam)
        last = Wk is W3
        h, xs = _lap_b_tc(h, t1, lap_agg(xs1), dinv8, lam, Wk[0], Wk[1],
                          Wk[2], bk.reshape(1, -1), with_xs=not last)

    return _readout_tc(gid3, h, mlp_w1, mlp_b1.reshape(1, -1), mlp_w2,
                       mlp_b2.reshape(1, -1))


# speedup vs baseline: 1.0005x; 1.0005x over previous
import jax
import jax.numpy as jnp
from jax import lax
from jax.experimental import pallas as pl
from jax.experimental.pallas import tpu as pltpu

N = 10000
E = 320000
B = 128
RBLK = 2000
NBLK = N // RBLK


def _prep_body(sig_ref, deg_ref, emb_ref, h0_ref, xs_ref, dinv_ref):
    sig = sig_ref[0, 0, :]
    oh = (sig[:, None] == lax.broadcasted_iota(jnp.int32, (RBLK, 32), 1))
    h0 = jnp.dot(oh.astype(jnp.float32), emb_ref[...],
                 preferred_element_type=jnp.float32)
    dinv8 = lax.rsqrt(jnp.maximum(deg_ref[...], 1.0))
    h0_ref[...] = h0
    dinv_ref[...] = dinv8
    xs_ref[...] = h0 * dinv8[:, 0:1]


def _prep_tc(signal3, deg8, emb_pad):
    f = 128
    return pl.pallas_call(
        _prep_body,
        grid=(NBLK,),
        in_specs=[
            pl.BlockSpec((1, 1, RBLK), lambda i: (i, 0, 0)),
            pl.BlockSpec((RBLK, 8), lambda i: (i, 0)),
            pl.BlockSpec((32, f), lambda i: (0, 0)),
        ],
        out_specs=[
            pl.BlockSpec((RBLK, f), lambda i: (i, 0)),
            pl.BlockSpec((RBLK, f), lambda i: (i, 0)),
            pl.BlockSpec((RBLK, 8), lambda i: (i, 0)),
        ],
        out_shape=[
            jax.ShapeDtypeStruct((N, f), jnp.float32),
            jax.ShapeDtypeStruct((N, f), jnp.float32),
            jax.ShapeDtypeStruct((N, 8), jnp.float32),
        ],
    )(signal3, deg8, emb_pad)


def _lap_a_body(h_ref, agg_ref, dinv_ref, lam_ref, t1_ref, xs_ref):
    h = h_ref[...]
    dv = dinv_ref[:, 0:1]
    rn = 2.0 / lam_ref[0, 0]
    t1 = rn * (h - agg_ref[...] * dv) - h
    t1_ref[...] = t1
    xs_ref[...] = t1 * dv


def _lap_a_tc(h, agg, dinv8, lam):
    f = h.shape[1]
    return pl.pallas_call(
        _lap_a_body,
        grid=(NBLK,),
        in_specs=[
            pl.BlockSpec((RBLK, f), lambda i: (i, 0)),
            pl.BlockSpec((RBLK, f), lambda i: (i, 0)),
            pl.BlockSpec((RBLK, 8), lambda i: (i, 0)),
            pl.BlockSpec((1, 1), lambda i: (0, 0)),
        ],
        out_specs=[pl.BlockSpec((RBLK, f), lambda i: (i, 0)),
                   pl.BlockSpec((RBLK, f), lambda i: (i, 0))],
        out_shape=[jax.ShapeDtypeStruct((N, f), jnp.float32),
                   jax.ShapeDtypeStruct((N, f), jnp.float32)],
    )(h, agg, dinv8, lam)


def _lap_b_body(h_ref, t1_ref, agg_ref, dinv_ref, lam_ref, w0_ref, w1_ref,
                w2_ref, b_ref, o_ref, xs_ref):
    h = h_ref[...]
    t1 = t1_ref[...]
    dv = dinv_ref[:, 0:1]
    rn = 2.0 / lam_ref[0, 0]
    t2 = 2.0 * (rn * (t1 - agg_ref[...] * dv) - t1) - h
    out = (jnp.dot(h, w0_ref[...], preferred_element_type=jnp.float32)
           + jnp.dot(t1, w1_ref[...], preferred_element_type=jnp.float32)
           + jnp.dot(t2, w2_ref[...], preferred_element_type=jnp.float32)
           + b_ref[...])
    o_ref[...] = out
    if xs_ref is not None:
        xs_ref[...] = out * dv


def _lap_b_tc(h, t1, agg, dinv8, lam, w0, w1, w2, b, with_xs):
    f = h.shape[1]
    fo = w0.shape[1]
    out_specs = [pl.BlockSpec((RBLK, fo), lambda i: (i, 0))]
    out_shape = [jax.ShapeDtypeStruct((N, fo), jnp.float32)]
    if with_xs:
        out_specs.append(pl.BlockSpec((RBLK, fo), lambda i: (i, 0)))
        out_shape.append(jax.ShapeDtypeStruct((N, fo), jnp.float32))
        body = _lap_b_body
    else:
        def body(*refs):
            _lap_b_body(*refs, None)

    res = pl.pallas_call(
        body,
        grid=(NBLK,),
        in_specs=[
            pl.BlockSpec((RBLK, f), lambda i: (i, 0)),
            pl.BlockSpec((RBLK, f), lambda i: (i, 0)),
            pl.BlockSpec((RBLK, f), lambda i: (i, 0)),
            pl.BlockSpec((RBLK, 8), lambda i: (i, 0)),
            pl.BlockSpec((1, 1), lambda i: (0, 0)),
            pl.BlockSpec((f, fo), lambda i: (0, 0)),
            pl.BlockSpec((f, fo), lambda i: (0, 0)),
            pl.BlockSpec((f, fo), lambda i: (0, 0)),
            pl.BlockSpec((1, fo), lambda i: (0, 0)),
        ],
        out_specs=out_specs,
        out_shape=out_shape,
    )(h, t1, agg, dinv8, lam, w0, w1, w2, b)
    if with_xs:
        return res[0], res[1]
    return res[0], None


def _readout_body(gid_ref, h_ref, w1_ref, b1_ref, w2_ref, b2_ref, o_ref, acc):
    i = pl.program_id(0)

    @pl.when(i == 0)
    def _():
        acc[...] = jnp.zeros_like(acc)

    g = gid_ref[0, 0, :]
    oh = (g[:, None] == lax.broadcasted_iota(jnp.int32, (RBLK, B), 1))
    acc[...] += lax.dot_general(oh.astype(jnp.float32), h_ref[...],
                                (((0,), (0,)), ((), ())),
                                preferred_element_type=jnp.float32)

    @pl.when(i == NBLK - 1)
    def _():
        z = jnp.maximum(jnp.dot(acc[...], w1_ref[...],
                                preferred_element_type=jnp.float32)
                        + b1_ref[...], 0.0)
        o_ref[...] = jnp.dot(z, w2_ref[...],
                             preferred_element_type=jnp.float32) + b2_ref[...]


def _readout_tc(gid3, h, w1, b1, w2, b2):
    f = h.shape[1]
    ncls = w2.shape[1]
    return pl.pallas_call(
        _readout_body,
        grid=(NBLK,),
        in_specs=[
            pl.BlockSpec((1, 1, RBLK), lambda i: (i, 0, 0)),
            pl.BlockSpec((RBLK, f), lambda i: (i, 0)),
            pl.BlockSpec((f, f), lambda i: (0, 0)),
            pl.BlockSpec((1, f), lambda i: (0, 0)),
            pl.BlockSpec((f, ncls), lambda i: (0, 0)),
            pl.BlockSpec((1, ncls), lambda i: (0, 0)),
        ],
        out_specs=pl.BlockSpec((B, ncls), lambda i: (0, 0)),
        out_shape=jax.ShapeDtypeStruct((B, ncls), jnp.float32),
        scratch_shapes=[pltpu.VMEM((B, f), jnp.float32)],
    )(gid3, h, w1, b1, w2, b2)


def kernel(signal, edge_index, node_graph_id, lambda_max, emb, W1, b1, W2, b2,
           W3, b3, mlp_w1, mlp_b1, mlp_w2, mlp_b2):
    src = edge_index[0].astype(jnp.int32)
    dst = edge_index[1].astype(jnp.int32)

    deg = jax.ops.segment_sum(jnp.ones((E,), jnp.float32), dst,
                              num_segments=N)
    deg8 = jnp.broadcast_to(deg[:, None], (N, 8))

    def lap_agg(xs):
        return jax.ops.segment_sum(jnp.take(xs, src, axis=0), dst,
                                   num_segments=N)

    signal3 = signal.astype(jnp.int32).reshape(NBLK, 1, RBLK)
    gid3 = node_graph_id.astype(jnp.int32).reshape(NBLK, 1, RBLK)
    emb_pad = jnp.zeros((32, 128), jnp.float32).at[:28].set(emb)
    lam = lambda_max.reshape(1, 1)

    h, xs, dinv8 = _prep_tc(signal3, deg8, emb_pad)

    for Wk, bk in ((W1, b1), (W2, b2), (W3, b3)):
        t1, xs1 = _lap_a_tc(h, lap_agg(xs), dinv8, lam)
        last = Wk is W3
        h, xs = _lap_b_tc(h, t1, lap_agg(xs1), dinv8, lam, Wk[0], Wk[1],
                          Wk[2], bk.reshape(1, -1), with_xs=not last)

    return _readout_tc(gid3, h, mlp_w1, mlp_b1.reshape(1, -1), mlp_w2,
                       mlp_b2.reshape(1, -1))
